# baseline (speedup 1.0000x reference)
import functools

import jax
import jax.numpy as jnp
from jax import lax
from jax.experimental import pallas as pl
from jax.experimental.pallas import tpu as pltpu
from jax.experimental.pallas import tpu_sc as plsc

VOCAB = 2548
DIM = 1000
BATCH = 4096


def _make_embedding_kernel():
    info = plsc.get_sparse_core_info()
    num_cores, num_subcores = info.num_cores, info.num_subcores
    num_workers = num_cores * num_subcores
    b_per_w = BATCH // num_workers

    mesh = plsc.VectorSubcoreMesh(core_axis_name="c", subcore_axis_name="s")

    @functools.partial(
        pl.kernel,
        mesh=mesh,
        out_type=jax.ShapeDtypeStruct((BATCH, DIM), jnp.float32),
        scratch_types=[
            pltpu.VMEM((b_per_w,), jnp.int32),
            pltpu.VMEM((b_per_w, DIM), jnp.float32),
            pltpu.SemaphoreType.DMA,
        ],
        compiler_params=pltpu.CompilerParams(use_tc_tiling_on_sc=False),
    )
    def emb(x_hbm, w_hbm, out_hbm, idx_v, rows_v, sem):
        wid = lax.axis_index("s") * num_cores + lax.axis_index("c")
        base = wid * b_per_w
        pltpu.sync_copy(x_hbm.at[pl.ds(base, b_per_w)], idx_v)
        pltpu.async_copy(w_hbm.at[idx_v], rows_v, sem).wait()
        pltpu.sync_copy(rows_v, out_hbm.at[pl.ds(base, b_per_w)])

    return emb


_emb = _make_embedding_kernel()


def kernel(x, weight):
    return _emb(x.astype(jnp.int32), weight)


# speedup vs baseline: 1.1994x; 1.1994x over previous
import functools

import jax
import jax.numpy as jnp
from jax import lax
from jax.experimental import pallas as pl
from jax.experimental.pallas import tpu as pltpu
from jax.experimental.pallas import tpu_sc as plsc

VOCAB = 2548
DIM = 1000
DIM_PAD = 1024
BATCH = 4096

CHUNK = 16
NBUF = 4


def _make_embedding_kernel():
    info = plsc.get_sparse_core_info()
    num_cores, num_subcores = info.num_cores, info.num_subcores
    num_workers = num_cores * num_subcores
    b_per_w = BATCH // num_workers
    nchunks = b_per_w // CHUNK

    mesh = plsc.VectorSubcoreMesh(core_axis_name="c", subcore_axis_name="s")

    @functools.partial(
        pl.kernel,
        mesh=mesh,
        out_type=jax.ShapeDtypeStruct((BATCH, DIM_PAD), jnp.float32),
        scratch_types=[
            pltpu.VMEM((b_per_w,), jnp.int32),
            [pltpu.VMEM((CHUNK, DIM_PAD), jnp.float32) for _ in range(NBUF)],
            [pltpu.SemaphoreType.DMA for _ in range(NBUF)],
            [pltpu.SemaphoreType.DMA for _ in range(NBUF)],
        ],
    )
    def emb(x_hbm, w_hbm, out_hbm, idx_v, bufs, gsems, wsems):
        wid = lax.axis_index("s") * num_cores + lax.axis_index("c")
        base = wid * b_per_w
        pltpu.sync_copy(x_hbm.at[pl.ds(base, b_per_w)], idx_v)

        def gather(c):
            return pltpu.async_copy(
                w_hbm.at[idx_v.at[pl.ds(c * CHUNK, CHUNK)]],
                bufs[c % NBUF],
                gsems[c % NBUF],
            )

        def write(c):
            return pltpu.async_copy(
                bufs[c % NBUF],
                out_hbm.at[pl.ds(base + c * CHUNK, CHUNK)],
                wsems[c % NBUF],
            )

        g, w = {}, {}
        lead = NBUF // 2
        for c in range(lead):
            g[c] = gather(c)
        for c in range(nchunks):
            if c >= lead:
                w[c - lead].wait()
            if c + lead < nchunks:
                g[c + lead] = gather(c + lead)
            g[c].wait()
            w[c] = write(c)
        for c in range(max(0, nchunks - lead), nchunks):
            w[c].wait()

    return emb


_emb = _make_embedding_kernel()


def kernel(x, weight):
    w_pad = jnp.pad(weight, ((0, 0), (0, DIM_PAD - DIM)))
    return _emb(x.astype(jnp.int32), w_pad)[:, :DIM]
